# baseline (speedup 1.0000x reference)
import functools

import jax
import jax.numpy as jnp
from jax import lax
from jax.experimental import pallas as pl
from jax.experimental.pallas import tpu as pltpu
from jax.experimental.pallas import tpu_sc as plsc

_NUM_EMB = 512
_EMB_DIM = 32
_COMMIT = 0.25
_ROWS = 16 * 64 * 64
_BLK = 4096
_GRID = _ROWS // _BLK

_NW = 32
_RPW = _ROWS // _NW
_RB = 64
_NCHUNK = _RPW // _RB


def _vq_body(xt_ref, sume_ref, embT_ref, emb_ref,
             idx_ref, out_ref, hist_ref, sse_ref):
    i = pl.program_id(0)
    xt = xt_ref[0]
    x = jnp.transpose(xt)
    sumx = jnp.sum(x * x, axis=1, keepdims=True)
    mm = jnp.dot(x, embT_ref[...], preferred_element_type=jnp.float32)
    t = sumx + sume_ref[...]
    dist = t - 2.0 * mm
    iota = jax.lax.broadcasted_iota(jnp.int32, (_BLK, _NUM_EMB), 1)
    m = jnp.min(dist, axis=1, keepdims=True)
    idx = jnp.min(jnp.where(dist == m, iota, _NUM_EMB), axis=1, keepdims=True)
    idx_ref[...] = idx
    enc = (iota == idx).astype(jnp.float32)
    qt = jax.lax.dot_general(emb_ref[...], enc, (((0,), (1,)), ((), ())),
                             preferred_element_type=jnp.float32)
    out_ref[0] = xt + (qt - xt)
    d = qt - xt
    part_hist = jnp.sum(enc, axis=0, keepdims=True)
    part_sse = jnp.sum(d * d)

    @pl.when(i == 0)
    def _init():
        hist_ref[...] = part_hist
        sse_ref[0, 0] = part_sse

    @pl.when(i != 0)
    def _acc():
        hist_ref[...] = hist_ref[...] + part_hist
        sse_ref[0, 0] = sse_ref[0, 0] + part_sse


def _sc_enc_body(idx_hbm, enc_hbm, idx_v, buf_v, sem):
    wid = lax.axis_index("s") * 2 + lax.axis_index("c")
    wbase = wid * _RPW
    iota16 = lax.iota(jnp.int32, 16)
    ones16 = jnp.full((16,), 1.0, jnp.float32)
    zeros16 = jnp.zeros((16,), jnp.float32)

    def _zero(i, carry):
        buf_v[pl.ds(i * 16, 16)] = zeros16
        return carry

    lax.fori_loop(0, (_RB * _NUM_EMB) // 16, _zero, 0)

    def _chunk(c, carry):
        base = wbase + c * _RB
        pltpu.sync_copy(idx_hbm.at[pl.ds(base, _RB)], idx_v)
        for g in range(_RB // 16):
            codes = idx_v[pl.ds(g * 16, 16)]
            flat = (g * 16 + iota16) * _NUM_EMB + codes
            plsc.store_scatter(buf_v, [flat], ones16)
        pltpu.sync_copy(buf_v, enc_hbm.at[pl.ds(base * _NUM_EMB,
                                                _RB * _NUM_EMB)])
        for g in range(_RB // 16):
            codes = idx_v[pl.ds(g * 16, 16)]
            flat = (g * 16 + iota16) * _NUM_EMB + codes
            plsc.store_scatter(buf_v, [flat], zeros16)
        return carry

    lax.fori_loop(0, _NCHUNK, _chunk, 0)


@functools.partial(jax.jit, static_argnames=())
def kernel(inputs, embedding):
    xt3 = inputs.reshape(16, _EMB_DIM, 64 * 64)
    sume = jnp.sum(embedding ** 2, axis=1).reshape(1, -1)
    embT = embedding.T

    n_sub = (64 * 64) // _BLK if _BLK <= 64 * 64 else 1
    blk_hw = min(_BLK, 64 * 64)

    idx2, out3, hist, sse = pl.pallas_call(
        _vq_body,
        grid=(_GRID,),
        in_specs=[
            pl.BlockSpec((1, _EMB_DIM, blk_hw),
                         lambda i: (i // n_sub, 0, i % n_sub)),
            pl.BlockSpec((1, _NUM_EMB), lambda i: (0, 0)),
            pl.BlockSpec((_EMB_DIM, _NUM_EMB), lambda i: (0, 0)),
            pl.BlockSpec((_NUM_EMB, _EMB_DIM), lambda i: (0, 0)),
        ],
        out_specs=[
            pl.BlockSpec((_BLK, 1), lambda i: (i, 0)),
            pl.BlockSpec((1, _EMB_DIM, blk_hw),
                         lambda i: (i // n_sub, 0, i % n_sub)),
            pl.BlockSpec((1, _NUM_EMB), lambda i: (0, 0)),
            pl.BlockSpec(memory_space=pltpu.SMEM, block_shape=(1, 1),
                         index_map=lambda i: (0, 0)),
        ],
        out_shape=[
            jax.ShapeDtypeStruct((_ROWS, 1), jnp.int32),
            jax.ShapeDtypeStruct((16, _EMB_DIM, 64 * 64), jnp.float32),
            jax.ShapeDtypeStruct((1, _NUM_EMB), jnp.float32),
            jax.ShapeDtypeStruct((1, 1), jnp.float32),
        ],
    )(xt3, sume, embT, embedding)

    sc_enc = functools.partial(
        pl.kernel,
        mesh=plsc.VectorSubcoreMesh(core_axis_name="c", subcore_axis_name="s"),
        out_type=jax.ShapeDtypeStruct((_ROWS * _NUM_EMB,), jnp.float32),
        scratch_types=[
            pltpu.VMEM((_RB,), jnp.int32),
            pltpu.VMEM((_RB * _NUM_EMB,), jnp.float32),
            pltpu.SemaphoreType.DMA,
        ],
        compiler_params=pltpu.CompilerParams(needs_layout_passes=False),
    )(_sc_enc_body)
    enc_flat = sc_enc(idx2.reshape(-1))
    enc = enc_flat.reshape(_ROWS, _NUM_EMB)

    n_el = _ROWS * _EMB_DIM
    mse = sse[0, 0] / n_el
    loss = mse + _COMMIT * mse
    out = out3.reshape(16, _EMB_DIM, 64, 64)
    avg_probs = hist[0] / _ROWS
    perplexity = jnp.exp(-jnp.sum(avg_probs * jnp.log(avg_probs + 1e-10)))
    return (loss, out, perplexity, enc)


# speedup vs baseline: 1.6054x; 1.6054x over previous
import functools

import jax
import jax.numpy as jnp
from jax import lax
from jax.experimental import pallas as pl
from jax.experimental.pallas import tpu as pltpu
from jax.experimental.pallas import tpu_sc as plsc

_NUM_EMB = 512
_EMB_DIM = 32
_COMMIT = 0.25
_ROWS = 16 * 64 * 64
_BLK = 4096
_GRID = _ROWS // _BLK

_NW = 32
_RPW = _ROWS // _NW
_RB = 64
_NCHUNK = _RPW // _RB


def _vq_body(xt_ref, sume_ref, embT_ref, emb_ref,
             idx_ref, out_ref, hist_ref, sse_ref):
    i = pl.program_id(0)
    xt = xt_ref[0]
    x = jnp.transpose(xt)
    sumx = jnp.sum(x * x, axis=1, keepdims=True)
    mm = jnp.dot(x, embT_ref[...], preferred_element_type=jnp.float32)
    t = sumx + sume_ref[...]
    dist = t - 2.0 * mm
    iota = jax.lax.broadcasted_iota(jnp.int32, (_BLK, _NUM_EMB), 1)
    m = jnp.min(dist, axis=1, keepdims=True)
    idx = jnp.min(jnp.where(dist == m, iota, _NUM_EMB), axis=1, keepdims=True)
    idx_ref[...] = idx
    enc = (iota == idx).astype(jnp.float32)
    qt = jax.lax.dot_general(emb_ref[...], enc, (((0,), (1,)), ((), ())),
                             preferred_element_type=jnp.float32)
    out_ref[0] = xt + (qt - xt)
    d = qt - xt
    part_hist = jnp.sum(enc, axis=0, keepdims=True)
    part_sse = jnp.sum(d * d)

    @pl.when(i == 0)
    def _init():
        hist_ref[...] = part_hist
        sse_ref[0, 0] = part_sse

    @pl.when(i != 0)
    def _acc():
        hist_ref[...] = hist_ref[...] + part_hist
        sse_ref[0, 0] = sse_ref[0, 0] + part_sse


def _sc_enc_body(idx_hbm, enc_hbm, idx_v, buf_v, sem):
    wid = lax.axis_index("s") * 2 + lax.axis_index("c")
    wbase = wid * _RPW
    iota16 = lax.iota(jnp.int32, 16)
    ones16 = jnp.full((16,), 1.0, jnp.float32)
    zeros16 = jnp.zeros((16,), jnp.float32)

    def _zero(i, carry):
        p = i * 16 + iota16
        plsc.store_scatter(buf_v, [p // _NUM_EMB, p % _NUM_EMB], zeros16)
        return carry


    lax.fori_loop(0, (_RB * _NUM_EMB) // 16, _zero, 0)

    def _scatter(vals):
        for g in range(_RB // 16):
            codes = idx_v[pl.ds(g * 16, 16)]
            r = g * 16 + iota16
            plsc.store_scatter(buf_v, [r, codes], vals)

    def _chunk(c, carry):
        base = wbase + c * _RB
        pltpu.sync_copy(idx_hbm.at[pl.ds(base, _RB)], idx_v)
        _scatter(ones16)
        pltpu.sync_copy(buf_v, enc_hbm.at[pl.ds(base, _RB)])
        _scatter(zeros16)
        return carry

    lax.fori_loop(0, _NCHUNK, _chunk, 0)


@functools.partial(jax.jit, static_argnames=())
def kernel(inputs, embedding):
    xt3 = inputs.reshape(16, _EMB_DIM, 64 * 64)
    sume = jnp.sum(embedding ** 2, axis=1).reshape(1, -1)
    embT = embedding.T

    n_sub = (64 * 64) // _BLK if _BLK <= 64 * 64 else 1
    blk_hw = min(_BLK, 64 * 64)

    idx2, out3, hist, sse = pl.pallas_call(
        _vq_body,
        grid=(_GRID,),
        in_specs=[
            pl.BlockSpec((1, _EMB_DIM, blk_hw),
                         lambda i: (i // n_sub, 0, i % n_sub)),
            pl.BlockSpec((1, _NUM_EMB), lambda i: (0, 0)),
            pl.BlockSpec((_EMB_DIM, _NUM_EMB), lambda i: (0, 0)),
            pl.BlockSpec((_NUM_EMB, _EMB_DIM), lambda i: (0, 0)),
        ],
        out_specs=[
            pl.BlockSpec((_BLK, 1), lambda i: (i, 0)),
            pl.BlockSpec((1, _EMB_DIM, blk_hw),
                         lambda i: (i // n_sub, 0, i % n_sub)),
            pl.BlockSpec((1, _NUM_EMB), lambda i: (0, 0)),
            pl.BlockSpec(memory_space=pltpu.SMEM, block_shape=(1, 1),
                         index_map=lambda i: (0, 0)),
        ],
        out_shape=[
            jax.ShapeDtypeStruct((_ROWS, 1), jnp.int32),
            jax.ShapeDtypeStruct((16, _EMB_DIM, 64 * 64), jnp.float32),
            jax.ShapeDtypeStruct((1, _NUM_EMB), jnp.float32),
            jax.ShapeDtypeStruct((1, 1), jnp.float32),
        ],
    )(xt3, sume, embT, embedding)

    sc_enc = functools.partial(
        pl.kernel,
        mesh=plsc.VectorSubcoreMesh(core_axis_name="c", subcore_axis_name="s"),
        out_type=jax.ShapeDtypeStruct((_ROWS, _NUM_EMB), jnp.float32),
        scratch_types=[
            pltpu.VMEM((_RB,), jnp.int32),
            pltpu.VMEM((_RB, _NUM_EMB), jnp.float32),
            pltpu.SemaphoreType.DMA,
        ],
        compiler_params=pltpu.CompilerParams(needs_layout_passes=False,
                                             use_tc_tiling_on_sc=True),
    )(_sc_enc_body)
    enc = sc_enc(idx2.reshape(-1))

    n_el = _ROWS * _EMB_DIM
    mse = sse[0, 0] / n_el
    loss = mse + _COMMIT * mse
    out = out3.reshape(16, _EMB_DIM, 64, 64)
    avg_probs = hist[0] / _ROWS
    perplexity = jnp.exp(-jnp.sum(avg_probs * jnp.log(avg_probs + 1e-10)))
    return (loss, out, perplexity, enc)


# speedup vs baseline: 2.6693x; 1.6627x over previous
import functools

import jax
import jax.numpy as jnp
from jax.experimental import pallas as pl
from jax.experimental.pallas import tpu as pltpu

_NUM_EMB = 512
_EMB_DIM = 32
_COMMIT = 0.25
_ROWS = 16 * 64 * 64
_BLK = 4096
_GRID = _ROWS // _BLK


def _vq_body(xt_ref, sume_ref, embT2_ref, emb_ref,
             enc_ref, out_ref, hist_ref, sse_ref):
    i = pl.program_id(0)
    xt = xt_ref[0]
    x = jnp.transpose(xt)
    sumx = jnp.sum(x * x, axis=1, keepdims=True)
    mm2 = jnp.dot(x, embT2_ref[...], preferred_element_type=jnp.float32)
    t = sumx + sume_ref[...]
    dist = t - mm2
    iota = jax.lax.broadcasted_iota(jnp.int32, (_BLK, _NUM_EMB), 1)
    m = jnp.min(dist, axis=1, keepdims=True)
    idx = jnp.min(jnp.where(dist == m, iota, _NUM_EMB), axis=1, keepdims=True)
    enc = (iota == idx).astype(jnp.float32)
    enc_ref[...] = enc
    qt = jax.lax.dot_general(emb_ref[...], enc, (((0,), (1,)), ((), ())),
                             preferred_element_type=jnp.float32)
    out_ref[0] = xt + (qt - xt)
    d = qt - xt
    part_hist = jnp.sum(enc, axis=0, keepdims=True)
    part_sse = jnp.sum(d * d)

    @pl.when(i == 0)
    def _init():
        hist_ref[...] = part_hist
        sse_ref[0, 0] = part_sse

    @pl.when(i != 0)
    def _acc():
        hist_ref[...] = hist_ref[...] + part_hist
        sse_ref[0, 0] = sse_ref[0, 0] + part_sse


@functools.partial(jax.jit, static_argnames=())
def kernel(inputs, embedding):
    xt3 = inputs.reshape(16, _EMB_DIM, 64 * 64)
    sume = jnp.sum(embedding ** 2, axis=1).reshape(1, -1)
    embT2 = embedding.T * 2.0

    n_sub = (64 * 64) // _BLK if _BLK <= 64 * 64 else 1
    blk_hw = min(_BLK, 64 * 64)

    enc, out3, hist, sse = pl.pallas_call(
        _vq_body,
        grid=(_GRID,),
        in_specs=[
            pl.BlockSpec((1, _EMB_DIM, blk_hw),
                         lambda i: (i // n_sub, 0, i % n_sub)),
            pl.BlockSpec((1, _NUM_EMB), lambda i: (0, 0)),
            pl.BlockSpec((_EMB_DIM, _NUM_EMB), lambda i: (0, 0)),
            pl.BlockSpec((_NUM_EMB, _EMB_DIM), lambda i: (0, 0)),
        ],
        out_specs=[
            pl.BlockSpec((_BLK, _NUM_EMB), lambda i: (i, 0)),
            pl.BlockSpec((1, _EMB_DIM, blk_hw),
                         lambda i: (i // n_sub, 0, i % n_sub)),
            pl.BlockSpec((1, _NUM_EMB), lambda i: (0, 0)),
            pl.BlockSpec(memory_space=pltpu.SMEM, block_shape=(1, 1),
                         index_map=lambda i: (0, 0)),
        ],
        out_shape=[
            jax.ShapeDtypeStruct((_ROWS, _NUM_EMB), jnp.float32),
            jax.ShapeDtypeStruct((16, _EMB_DIM, 64 * 64), jnp.float32),
            jax.ShapeDtypeStruct((1, _NUM_EMB), jnp.float32),
            jax.ShapeDtypeStruct((1, 1), jnp.float32),
        ],
    )(xt3, sume, embT2, embedding)

    n_el = _ROWS * _EMB_DIM
    mse = sse[0, 0] / n_el
    loss = mse + _COMMIT * mse
    out = out3.reshape(16, _EMB_DIM, 64, 64)
    avg_probs = hist[0] / _ROWS
    perplexity = jnp.exp(-jnp.sum(avg_probs * jnp.log(avg_probs + 1e-10)))
    return (loss, out, perplexity, enc)
